# Initial kernel scaffold; baseline (speedup 1.0000x reference)
import jax
import jax.numpy as jnp
from jax.experimental import pallas as pl


def kernel(triangles, points):
    raise NotImplementedError("write your pallas kernel here")


# speedup vs baseline: 3.0474x; 3.0474x over previous
import functools

import jax
import jax.numpy as jnp
from jax import lax
from jax.experimental import pallas as pl
from jax.experimental.pallas import tpu as pltpu
from jax.experimental.pallas import tpu_sc as plsc

F = 4096
Q = 4096
NC, NS, L = 2, 16, 16
NW = NC * NS
PPW = Q // NW
NG = PPW // L

_EPS = 1e-12


def _safe_div(num, den):
    return num / jnp.where(jnp.abs(den) > _EPS, den, 1.0)


def _closest_from_rows(rows, px, py, pz):
    ax, ay, az, bx, by, bz, cx, cy, cz, abx, aby, abz, acx, acy, acz = rows
    apx, apy, apz = px - ax, py - ay, pz - az
    d1 = abx * apx + aby * apy + abz * apz
    d2 = acx * apx + acy * apy + acz * apz
    bpx, bpy, bpz = px - bx, py - by, pz - bz
    d3 = abx * bpx + aby * bpy + abz * bpz
    d4 = acx * bpx + acy * bpy + acz * bpz
    cpx, cpy, cpz = px - cx, py - cy, pz - cz
    d5 = abx * cpx + aby * cpy + abz * cpz
    d6 = acx * cpx + acy * cpy + acz * cpz
    vc = d1 * d4 - d3 * d2
    vb = d5 * d2 - d1 * d6
    va = d3 * d6 - d5 * d4
    t_ab = _safe_div(d1, d1 - d3)
    t_ac = _safe_div(d2, d2 - d6)
    t_bc = _safe_div(d4 - d3, (d4 - d3) + (d5 - d6))
    denom = va + vb + vc
    v_face = _safe_div(vb, denom)
    w_face = _safe_div(vc, denom)
    m1 = (d1 <= 0) & (d2 <= 0)
    m2 = (d3 >= 0) & (d4 <= d3)
    m3 = (vc <= 0) & (d1 >= 0) & (d3 <= 0)
    m4 = (d6 >= 0) & (d5 <= d6)
    m5 = (vb <= 0) & (d2 >= 0) & (d6 <= 0)
    m6 = (va <= 0) & ((d4 - d3) >= 0) & ((d5 - d6) >= 0)
    zero = jnp.zeros_like(d1)
    one = jnp.ones_like(d1)

    def _select(cases, default):
        out = default
        for m, val in reversed(cases):
            out = jnp.where(m, val, out)
        return out

    v = _select([(m1, zero), (m2, one), (m3, t_ab), (m4, zero),
                 (m5, zero), (m6, 1.0 - t_bc)], v_face)
    w = _select([(m1, zero), (m2, zero), (m3, zero), (m4, one),
                 (m5, t_ac), (m6, t_bc)], w_face)
    clx = ax + v * abx + w * acx
    cly = ay + v * aby + w * acy
    clz = az + v * abz + w * acz
    dx, dy, dz = px - clx, py - cly, pz - clz
    dist2 = dx * dx + dy * dy + dz * dz
    return dist2, clx, cly, clz


def _sc_body(face_hbm, pts_hbm, out_d, out_c, out_f,
             face_v, pts_v, dist_v, clos_v, bidx_v):
    wid = lax.axis_index("s") * NC + lax.axis_index("c")
    base = wid * PPW
    pltpu.sync_copy(face_hbm, face_v)
    pltpu.sync_copy(pts_hbm.at[:, pl.ds(base, PPW)], pts_v)

    for g in range(NG):
        sl = pl.ds(g * L, L)
        px = pts_v[0, sl]
        py = pts_v[1, sl]
        pz = pts_v[2, sl]

        def scan_face(f, carry):
            bd, bi = carry
            ffull = jnp.full((L,), f, jnp.int32)
            rows = [plsc.load_gather(face_v, [ffull + (r * F)])
                    for r in range(15)]
            dist2, _, _, _ = _closest_from_rows(rows, px, py, pz)
            m = dist2 < bd
            bd = jnp.where(m, dist2, bd)
            bi = jnp.where(m, ffull, bi)
            return bd, bi

        init = (jnp.full((L,), jnp.inf, jnp.float32),
                jnp.zeros((L,), jnp.int32))
        bd, bi = lax.fori_loop(0, F, scan_face, init)

        rows = [plsc.load_gather(face_v, [bi + (r * F)])
                for r in range(15)]
        dist2, clx, cly, clz = _closest_from_rows(rows, px, py, pz)
        dist_v[sl] = dist2
        bidx_v[sl] = bi
        clos_v[0, sl] = clx
        clos_v[1, sl] = cly
        clos_v[2, sl] = clz

    pltpu.sync_copy(dist_v, out_d.at[pl.ds(base, PPW)])
    pltpu.sync_copy(bidx_v, out_f.at[pl.ds(base, PPW)])
    pltpu.sync_copy(clos_v, out_c.at[:, pl.ds(base, PPW)])


@functools.cache
def _sc_call():
    return functools.partial(
        pl.kernel,
        out_type=(
            jax.ShapeDtypeStruct((Q,), jnp.float32),
            jax.ShapeDtypeStruct((3, Q), jnp.float32),
            jax.ShapeDtypeStruct((Q,), jnp.int32),
        ),
        mesh=plsc.VectorSubcoreMesh(
            core_axis_name="c", subcore_axis_name="s",
            num_cores=NC, num_subcores=NS),
        scratch_types=[
            pltpu.VMEM((15 * F,), jnp.float32),
            pltpu.VMEM((3, PPW), jnp.float32),
            pltpu.VMEM((PPW,), jnp.float32),
            pltpu.VMEM((3, PPW), jnp.float32),
            pltpu.VMEM((PPW,), jnp.int32),
        ],
        compiler_params=pltpu.CompilerParams(use_tc_tiling_on_sc=False,
                                             needs_layout_passes=False),
    )(_sc_body)


def kernel(triangles, points):
    tri = triangles[0]
    a = tri[:, 0, :]
    b = tri[:, 1, :]
    c = tri[:, 2, :]
    face = jnp.concatenate(
        [a.T, b.T, c.T, (b - a).T, (c - a).T], axis=0).reshape(-1)
    pts = points[0].T
    d, cl, fi = _sc_call()(face, pts)
    return d[None], cl.T[None], fi[None]
